# baseline (speedup 1.0000x reference)
import functools

import jax
import jax.numpy as jnp
from jax import lax
from jax.experimental import pallas as pl
from jax.experimental.pallas import tpu as pltpu
from jax.experimental.pallas import tpu_sc as plsc

_TOPK = 2
_BM = 256
_BK = 256
_NC = 2
_NS = 16
_NWRK = _NC * _NS


def _gate_body(x_ref, wg_ref, i1_ref, i2_ref, w1_ref, w2_ref):
    x = x_ref[...]
    logits = lax.dot_general(x, wg_ref[...], (((1,), (1,)), ((), ())),
                             preferred_element_type=jnp.float32)
    m = jnp.max(logits, axis=-1, keepdims=True)
    ex = jnp.exp(logits - m)
    p = ex / jnp.sum(ex, axis=-1, keepdims=True)
    e_num = p.shape[-1]
    idxs = lax.broadcasted_iota(jnp.int32, p.shape, 1)
    m1 = jnp.max(p, axis=-1, keepdims=True)
    a1 = jnp.min(jnp.where(p == m1, idxs, e_num), axis=-1, keepdims=True)
    p2 = jnp.where(idxs == a1, -1.0, p)
    m2 = jnp.max(p2, axis=-1, keepdims=True)
    a2 = jnp.min(jnp.where(p2 == m2, idxs, e_num), axis=-1, keepdims=True)
    s = m1 + m2 + 1e-20
    i1_ref[...] = a1
    i2_ref[...] = a2
    w1_ref[...] = jnp.broadcast_to(m1 / s, w1_ref.shape)
    w2_ref[...] = jnp.broadcast_to(m2 / s, w2_ref.shape)


def _gate(xf, Wg):
    t = xf.shape[0]
    return pl.pallas_call(
        _gate_body,
        out_shape=[
            jax.ShapeDtypeStruct((t, 1), jnp.int32),
            jax.ShapeDtypeStruct((t, 1), jnp.int32),
            jax.ShapeDtypeStruct((t, 16), jnp.float32),
            jax.ShapeDtypeStruct((t, 16), jnp.float32),
        ],
    )(xf, Wg)


def _dispatch(xf, tok_ids):
    t, h = xf.shape
    m = tok_ids.shape[0]
    spw = m // _NWRK
    nch = 4
    ch = spw // nch
    mesh = plsc.VectorSubcoreMesh(core_axis_name="c", subcore_axis_name="s")

    @functools.partial(
        pl.kernel, mesh=mesh,
        out_type=jax.ShapeDtypeStruct((m, h), jnp.float32),
        scratch_types=(
            [pltpu.VMEM((ch,), jnp.int32)] * nch
            + [pltpu.VMEM((ch, h), jnp.float32)] * 2
            + [pltpu.SemaphoreType.DMA] * 2
        ),
    )
    def k(xf_hbm, tok_hbm, xs_hbm, *refs):
        toks = refs[:nch]
        bufs = refs[nch:nch + 2]
        sems = refs[nch + 2:]
        wid = lax.axis_index("s") * _NC + lax.axis_index("c")
        base = wid * spw
        for c in range(nch):
            pltpu.sync_copy(tok_hbm.at[pl.ds(base + c * ch, ch)], toks[c])
        cps = [pltpu.async_copy(xf_hbm.at[toks[0]], bufs[0], sems[0]),
               pltpu.async_copy(xf_hbm.at[toks[1]], bufs[1], sems[1]),
               None, None]
        for c in range(nch):
            cps[c].wait()
            pltpu.sync_copy(bufs[c % 2], xs_hbm.at[pl.ds(base + c * ch, ch)])
            if c + 2 < nch:
                cps[c + 2] = pltpu.async_copy(
                    xf_hbm.at[toks[c + 2]], bufs[c % 2], sems[c % 2])

    return k(xf, tok_ids)


def _combine(ys, p0, p1, w1b, w2b):
    m, h = ys.shape
    t = p0.shape[0]
    tpw = t // _NWRK
    half = tpw // 2
    mesh = plsc.VectorSubcoreMesh(core_axis_name="c", subcore_axis_name="s")

    @functools.partial(
        pl.kernel, mesh=mesh,
        out_type=jax.ShapeDtypeStruct((t, h), jnp.float32),
        scratch_types=[
            pltpu.VMEM((half,), jnp.int32),
            pltpu.VMEM((half,), jnp.int32),
            pltpu.VMEM((half, 16), jnp.float32),
            pltpu.VMEM((half, 16), jnp.float32),
            pltpu.VMEM((half, h), jnp.float32),
            pltpu.VMEM((half, h), jnp.float32),
            pltpu.SemaphoreType.DMA,
            pltpu.SemaphoreType.DMA,
        ],
    )
    def k(ys_hbm, p0_hbm, p1_hbm, w1_hbm, w2_hbm, y_hbm,
          pa, pb, w1_v, w2_v, a_v, b_v, sem0, sem1):
        wid = lax.axis_index("s") * _NC + lax.axis_index("c")
        base = wid * tpw

        def do_half(off):
            pltpu.sync_copy(p0_hbm.at[pl.ds(base + off, half)], pa)
            pltpu.sync_copy(p1_hbm.at[pl.ds(base + off, half)], pb)
            pltpu.sync_copy(w1_hbm.at[pl.ds(base + off, half)], w1_v)
            pltpu.sync_copy(w2_hbm.at[pl.ds(base + off, half)], w2_v)
            cp0 = pltpu.async_copy(ys_hbm.at[pa], a_v, sem0)
            cp1 = pltpu.async_copy(ys_hbm.at[pb], b_v, sem1)
            cp0.wait()
            cp1.wait()

            def row(r, carry):
                wa = w1_v[r, :]
                wb = w2_v[r, :]
                for c in range(h // 16):
                    sl = pl.ds(c * 16, 16)
                    a_v[r, sl] = a_v[r, sl] * wa + b_v[r, sl] * wb
                return carry

            lax.fori_loop(0, half, row, 0)
            pltpu.sync_copy(a_v, y_hbm.at[pl.ds(base + off, half)])

        do_half(0)
        do_half(half)

    return k(ys, p0, p1, w1b, w2b)


def _gmm_body(nk, e_ref, b_ref, xs_ref, wg_ref, wu_ref, wd_ref, out_ref,
              g_acc, u_acc):
    k = pl.program_id(1)

    xk = xs_ref[...]
    gp = jnp.dot(xk, wg_ref[0], preferred_element_type=jnp.float32)
    up = jnp.dot(xk, wu_ref[0], preferred_element_type=jnp.float32)

    @pl.when(k == 0)
    def _():
        g_acc[...] = gp
        u_acc[...] = up

    @pl.when(k != 0)
    def _():
        g_acc[...] += gp
        u_acc[...] += up

    @pl.when(k == nk - 1)
    def _():
        g = g_acc[...]
        hh = (g * jax.nn.sigmoid(g)) * u_acc[...]
        out_ref[...] = jnp.dot(hh, wd_ref[0],
                               preferred_element_type=jnp.float32)


def _gmm(xs, Wgate, Wup, Wdown, wi_e, wi_b, nw):
    m, h = xs.shape
    ff = Wgate.shape[2]
    nk = h // _BK
    grid_spec = pltpu.PrefetchScalarGridSpec(
        num_scalar_prefetch=2,
        grid=(nw, nk),
        in_specs=[
            pl.BlockSpec((_BM, _BK), lambda i, k, e, b: (b[i], k)),
            pl.BlockSpec((1, _BK, ff), lambda i, k, e, b: (e[i], k, 0)),
            pl.BlockSpec((1, _BK, ff), lambda i, k, e, b: (e[i], k, 0)),
            pl.BlockSpec((1, ff, h), lambda i, k, e, b: (e[i], 0, 0)),
        ],
        out_specs=pl.BlockSpec((_BM, h), lambda i, k, e, b: (i, 0)),
        scratch_shapes=[
            pltpu.VMEM((_BM, ff), jnp.float32),
            pltpu.VMEM((_BM, ff), jnp.float32),
        ],
    )
    return pl.pallas_call(
        functools.partial(_gmm_body, nk),
        grid_spec=grid_spec,
        out_shape=jax.ShapeDtypeStruct((nw * _BM, h), jnp.float32),
        compiler_params=pltpu.CompilerParams(
            dimension_semantics=("arbitrary", "arbitrary")),
    )(wi_e, wi_b, xs, Wgate, Wup, Wdown)


def kernel(x, Wg, Wgate, Wup, Wdown):
    b, s, h = x.shape
    e_num = Wg.shape[0]
    xf = x.reshape(-1, h)
    t = xf.shape[0]
    m = t * _TOPK
    nb = m // _BM
    nw = nb + e_num - 1

    a1, a2, w1b, w2b = _gate(xf, Wg)
    eid = jnp.concatenate([a1, a2], axis=1).reshape(-1)

    oh = (eid[:, None] == jnp.arange(e_num)[None, :]).astype(jnp.int32)
    inc = jnp.cumsum(oh, axis=0)
    counts = inc[-1]
    off = jnp.concatenate([jnp.zeros((1,), jnp.int32), jnp.cumsum(counts)])
    rank = jnp.sum((inc - oh) * oh, axis=1)
    ip = (off[eid] + rank).astype(jnp.int32)
    tok_ids = jnp.zeros((m,), jnp.int32).at[ip].set(
        (jnp.arange(m, dtype=jnp.int32) // _TOPK))

    starts_e, ends_e = off[:e_num], off[1:]
    b_start = jnp.arange(nb, dtype=jnp.int32) * _BM
    overlap = ((starts_e[:, None] < (b_start + _BM)[None, :])
               & (ends_e[:, None] > b_start[None, :])
               & (counts[:, None] > 0))
    sel = jnp.nonzero(overlap.reshape(-1), size=nw,
                      fill_value=nb * e_num - 1)[0].astype(jnp.int32)
    wi_e = sel // nb
    wi_b = sel % nb

    fb = starts_e // _BM
    lb = jnp.maximum(ends_e - 1, 0) // _BM
    nblk = jnp.where(counts > 0, lb - fb + 1, 0)
    base_items = jnp.cumsum(nblk) - nblk
    item = base_items[eid] + (ip // _BM - fb[eid])
    ep = (item * _BM + ip % _BM).astype(jnp.int32)
    p0, p1 = ep[0::2], ep[1::2]

    xs = _dispatch(xf, tok_ids)

    ys = _gmm(xs, Wgate, Wup, Wdown, wi_e, wi_b, nw)

    y = _combine(ys, p0, p1, w1b, w2b)
    return y.reshape(b, s, h)


# speedup vs baseline: 1.4200x; 1.4200x over previous
import functools

import jax
import jax.numpy as jnp
from jax import lax
from jax.experimental import pallas as pl
from jax.experimental.pallas import tpu as pltpu
from jax.experimental.pallas import tpu_sc as plsc

_TOPK = 2
_BM = 256
_BK = 256
_NC = 2
_NS = 16
_NWRK = _NC * _NS


def _gate_body(x_ref, wg_ref, i1_ref, i2_ref, w1_ref, w2_ref):
    x = x_ref[...]
    logits = lax.dot_general(x, wg_ref[...], (((1,), (1,)), ((), ())),
                             preferred_element_type=jnp.float32)
    m = jnp.max(logits, axis=-1, keepdims=True)
    ex = jnp.exp(logits - m)
    p = ex / jnp.sum(ex, axis=-1, keepdims=True)
    e_num = p.shape[-1]
    idxs = lax.broadcasted_iota(jnp.int32, p.shape, 1)
    m1 = jnp.max(p, axis=-1, keepdims=True)
    a1 = jnp.min(jnp.where(p == m1, idxs, e_num), axis=-1, keepdims=True)
    p2 = jnp.where(idxs == a1, -1.0, p)
    m2 = jnp.max(p2, axis=-1, keepdims=True)
    a2 = jnp.min(jnp.where(p2 == m2, idxs, e_num), axis=-1, keepdims=True)
    s = m1 + m2 + 1e-20
    i1_ref[...] = a1
    i2_ref[...] = a2
    w1_ref[...] = jnp.broadcast_to(m1 / s, w1_ref.shape)
    w2_ref[...] = jnp.broadcast_to(m2 / s, w2_ref.shape)


def _gate(xf, Wg):
    t = xf.shape[0]
    return pl.pallas_call(
        _gate_body,
        out_shape=[
            jax.ShapeDtypeStruct((t, 1), jnp.int32),
            jax.ShapeDtypeStruct((t, 1), jnp.int32),
            jax.ShapeDtypeStruct((t, 16), jnp.float32),
            jax.ShapeDtypeStruct((t, 16), jnp.float32),
        ],
    )(xf, Wg)


def _dispatch(xf, tok_ids):
    t, h = xf.shape
    m = tok_ids.shape[0]
    spw = m // _NWRK
    nch = 4
    ch = spw // nch
    mesh = plsc.VectorSubcoreMesh(core_axis_name="c", subcore_axis_name="s")

    @functools.partial(
        pl.kernel, mesh=mesh,
        out_type=jax.ShapeDtypeStruct((m, h), jnp.float32),
        scratch_types=(
            [pltpu.VMEM((ch,), jnp.int32)] * nch
            + [pltpu.VMEM((ch, h), jnp.float32)] * 2
            + [pltpu.SemaphoreType.DMA] * 2
        ),
    )
    def k(xf_hbm, tok_hbm, xs_hbm, *refs):
        toks = refs[:nch]
        bufs = refs[nch:nch + 2]
        sems = refs[nch + 2:]
        wid = lax.axis_index("s") * _NC + lax.axis_index("c")
        base = wid * spw
        for c in range(nch):
            pltpu.sync_copy(tok_hbm.at[pl.ds(base + c * ch, ch)], toks[c])
        cps = [pltpu.async_copy(xf_hbm.at[toks[0]], bufs[0], sems[0]),
               pltpu.async_copy(xf_hbm.at[toks[1]], bufs[1], sems[1]),
               None, None]
        for c in range(nch):
            cps[c].wait()
            pltpu.sync_copy(bufs[c % 2], xs_hbm.at[pl.ds(base + c * ch, ch)])
            if c + 2 < nch:
                cps[c + 2] = pltpu.async_copy(
                    xf_hbm.at[toks[c + 2]], bufs[c % 2], sems[c % 2])

    return k(xf, tok_ids)


def _combine(ys, p0, p1, w1b, w2b):
    m, h = ys.shape
    t = p0.shape[0]
    tpw = t // _NWRK
    half = tpw // 2
    mesh = plsc.VectorSubcoreMesh(core_axis_name="c", subcore_axis_name="s")

    @functools.partial(
        pl.kernel, mesh=mesh,
        out_type=jax.ShapeDtypeStruct((t, h), jnp.float32),
        scratch_types=[
            pltpu.VMEM((half,), jnp.int32),
            pltpu.VMEM((half,), jnp.int32),
            pltpu.VMEM((half, 16), jnp.float32),
            pltpu.VMEM((half, 16), jnp.float32),
            pltpu.VMEM((half, h), jnp.float32),
            pltpu.VMEM((half, h), jnp.float32),
            pltpu.SemaphoreType.DMA,
            pltpu.SemaphoreType.DMA,
        ],
    )
    def k(ys_hbm, p0_hbm, p1_hbm, w1_hbm, w2_hbm, y_hbm,
          pa, pb, w1_v, w2_v, a_v, b_v, sem0, sem1):
        wid = lax.axis_index("s") * _NC + lax.axis_index("c")
        base = wid * tpw

        def do_half(off):
            pltpu.sync_copy(p0_hbm.at[pl.ds(base + off, half)], pa)
            pltpu.sync_copy(p1_hbm.at[pl.ds(base + off, half)], pb)
            pltpu.sync_copy(w1_hbm.at[pl.ds(base + off, half)], w1_v)
            pltpu.sync_copy(w2_hbm.at[pl.ds(base + off, half)], w2_v)
            cp0 = pltpu.async_copy(ys_hbm.at[pa], a_v, sem0)
            cp1 = pltpu.async_copy(ys_hbm.at[pb], b_v, sem1)
            cp0.wait()
            cp1.wait()

            def row(r, carry):
                wa = w1_v[r, :]
                wb = w2_v[r, :]
                for c in range(h // 16):
                    sl = pl.ds(c * 16, 16)
                    a_v[r, sl] = a_v[r, sl] * wa + b_v[r, sl] * wb
                return carry

            lax.fori_loop(0, half, row, 0)
            pltpu.sync_copy(a_v, y_hbm.at[pl.ds(base + off, half)])

        do_half(0)
        do_half(half)

    return k(ys, p0, p1, w1b, w2b)


def _gmm_body(nk, e_ref, b_ref, xs_ref, wg_ref, wu_ref, wd_ref, out_ref,
              g_acc, u_acc):
    k = pl.program_id(1)

    xk = xs_ref[...]
    gp = lax.dot_general(xk, wg_ref[0], (((1,), (1,)), ((), ())),
                         preferred_element_type=jnp.float32)
    up = lax.dot_general(xk, wu_ref[0], (((1,), (1,)), ((), ())),
                         preferred_element_type=jnp.float32)

    @pl.when(k == 0)
    def _():
        g_acc[...] = gp
        u_acc[...] = up

    @pl.when(k != 0)
    def _():
        g_acc[...] += gp
        u_acc[...] += up

    @pl.when(k == nk - 1)
    def _():
        g = g_acc[...]
        hh = (g * jax.nn.sigmoid(g)) * u_acc[...]
        out_ref[...] = jnp.dot(hh, wd_ref[0],
                               preferred_element_type=jnp.float32)


def _gmm(xs, Wgate_t, Wup_t, Wdown, wi_e, wi_b, nw):
    m, h = xs.shape
    ff = Wgate_t.shape[1]
    nk = h // _BK
    grid_spec = pltpu.PrefetchScalarGridSpec(
        num_scalar_prefetch=2,
        grid=(nw, nk),
        in_specs=[
            pl.BlockSpec((_BM, _BK), lambda i, k, e, b: (b[i], k)),
            pl.BlockSpec((1, ff, _BK), lambda i, k, e, b: (e[i], 0, k)),
            pl.BlockSpec((1, ff, _BK), lambda i, k, e, b: (e[i], 0, k)),
            pl.BlockSpec((1, ff, h), lambda i, k, e, b: (e[i], 0, 0)),
        ],
        out_specs=pl.BlockSpec((_BM, h), lambda i, k, e, b: (i, 0)),
        scratch_shapes=[
            pltpu.VMEM((_BM, ff), jnp.float32),
            pltpu.VMEM((_BM, ff), jnp.float32),
        ],
    )
    return pl.pallas_call(
        functools.partial(_gmm_body, nk),
        grid_spec=grid_spec,
        out_shape=jax.ShapeDtypeStruct((nw * _BM, h), jnp.float32),
        compiler_params=pltpu.CompilerParams(
            dimension_semantics=("arbitrary", "arbitrary")),
    )(wi_e, wi_b, xs, Wgate_t, Wup_t, Wdown)


def kernel(x, Wg, Wgate, Wup, Wdown):
    b, s, h = x.shape
    e_num = Wg.shape[0]
    xf = x.reshape(-1, h)
    t = xf.shape[0]
    m = t * _TOPK
    nb = m // _BM
    nw = nb + e_num - 1

    a1, a2, w1b, w2b = _gate(xf, Wg)
    eid = jnp.concatenate([a1, a2], axis=1).reshape(-1)

    oh = (eid[:, None] == jnp.arange(e_num)[None, :]).astype(jnp.int32)
    inc = jnp.cumsum(oh, axis=0)
    counts = inc[-1]
    off = jnp.concatenate([jnp.zeros((1,), jnp.int32), jnp.cumsum(counts)])
    rank = jnp.sum((inc - oh) * oh, axis=1)
    ip = (off[eid] + rank).astype(jnp.int32)
    tok_ids = jnp.zeros((m,), jnp.int32).at[ip].set(
        (jnp.arange(m, dtype=jnp.int32) // _TOPK))

    starts_e, ends_e = off[:e_num], off[1:]
    b_start = jnp.arange(nb, dtype=jnp.int32) * _BM
    overlap = ((starts_e[:, None] < (b_start + _BM)[None, :])
               & (ends_e[:, None] > b_start[None, :])
               & (counts[:, None] > 0))
    sel = jnp.nonzero(overlap.reshape(-1), size=nw,
                      fill_value=nb * e_num - 1)[0].astype(jnp.int32)
    wi_e = sel // nb
    wi_b = sel % nb

    fb = starts_e // _BM
    lb = jnp.maximum(ends_e - 1, 0) // _BM
    nblk = jnp.where(counts > 0, lb - fb + 1, 0)
    base_items = jnp.cumsum(nblk) - nblk
    item = base_items[eid] + (ip // _BM - fb[eid])
    ep = (item * _BM + ip % _BM).astype(jnp.int32)
    p0, p1 = ep[0::2], ep[1::2]

    xs = _dispatch(xf, tok_ids)

    ys = _gmm(xs, jnp.swapaxes(Wgate, 1, 2), jnp.swapaxes(Wup, 1, 2),
              Wdown, wi_e, wi_b, nw)

    y = _combine(ys, p0, p1, w1b, w2b)
    return y.reshape(b, s, h)


# speedup vs baseline: 1.5005x; 1.0567x over previous
import functools

import jax
import jax.numpy as jnp
from jax import lax
from jax.experimental import pallas as pl
from jax.experimental.pallas import tpu as pltpu
from jax.experimental.pallas import tpu_sc as plsc

_TOPK = 2
_BM = 256
_BFF = 1408
_NC = 2
_NS = 16
_NWRK = _NC * _NS


def _gate_body(x_ref, wg_ref, i1_ref, i2_ref, w1_ref, w2_ref):
    x = x_ref[...]
    logits = lax.dot_general(x, wg_ref[...], (((1,), (1,)), ((), ())),
                             preferred_element_type=jnp.float32)
    m = jnp.max(logits, axis=-1, keepdims=True)
    ex = jnp.exp(logits - m)
    p = ex / jnp.sum(ex, axis=-1, keepdims=True)
    e_num = p.shape[-1]
    idxs = lax.broadcasted_iota(jnp.int32, p.shape, 1)
    m1 = jnp.max(p, axis=-1, keepdims=True)
    a1 = jnp.min(jnp.where(p == m1, idxs, e_num), axis=-1, keepdims=True)
    p2 = jnp.where(idxs == a1, -1.0, p)
    m2 = jnp.max(p2, axis=-1, keepdims=True)
    a2 = jnp.min(jnp.where(p2 == m2, idxs, e_num), axis=-1, keepdims=True)
    s = m1 + m2 + 1e-20
    i1_ref[...] = a1
    i2_ref[...] = a2
    w1_ref[...] = jnp.broadcast_to(m1 / s, w1_ref.shape)
    w2_ref[...] = jnp.broadcast_to(m2 / s, w2_ref.shape)


def _gate(xf, Wg):
    t = xf.shape[0]
    return pl.pallas_call(
        _gate_body,
        out_shape=[
            jax.ShapeDtypeStruct((t, 1), jnp.int32),
            jax.ShapeDtypeStruct((t, 1), jnp.int32),
            jax.ShapeDtypeStruct((t, 16), jnp.float32),
            jax.ShapeDtypeStruct((t, 16), jnp.float32),
        ],
    )(xf, Wg)


def _dispatch(xf, tok_ids):
    t, h = xf.shape
    m = tok_ids.shape[0]
    spw = m // _NWRK
    nch = 4
    ch = spw // nch
    mesh = plsc.VectorSubcoreMesh(core_axis_name="c", subcore_axis_name="s")

    @functools.partial(
        pl.kernel, mesh=mesh,
        out_type=jax.ShapeDtypeStruct((m, h), jnp.float32),
        scratch_types=(
            [pltpu.VMEM((ch,), jnp.int32)] * nch
            + [pltpu.VMEM((ch, h), jnp.float32)] * 2
            + [pltpu.SemaphoreType.DMA] * 2
        ),
    )
    def k(xf_hbm, tok_hbm, xs_hbm, *refs):
        toks = refs[:nch]
        bufs = refs[nch:nch + 2]
        sems = refs[nch + 2:]
        wid = lax.axis_index("s") * _NC + lax.axis_index("c")
        base = wid * spw
        for c in range(nch):
            pltpu.sync_copy(tok_hbm.at[pl.ds(base + c * ch, ch)], toks[c])
        cps = [pltpu.async_copy(xf_hbm.at[toks[0]], bufs[0], sems[0]),
               pltpu.async_copy(xf_hbm.at[toks[1]], bufs[1], sems[1]),
               None, None]
        for c in range(nch):
            cps[c].wait()
            pltpu.sync_copy(bufs[c % 2], xs_hbm.at[pl.ds(base + c * ch, ch)])
            if c + 2 < nch:
                cps[c + 2] = pltpu.async_copy(
                    xf_hbm.at[toks[c + 2]], bufs[c % 2], sems[c % 2])

    return k(xf, tok_ids)


def _combine(ys, p0, p1, w1b, w2b):
    m, h = ys.shape
    t = p0.shape[0]
    tpw = t // _NWRK
    half = tpw // 2
    mesh = plsc.VectorSubcoreMesh(core_axis_name="c", subcore_axis_name="s")

    @functools.partial(
        pl.kernel, mesh=mesh,
        out_type=jax.ShapeDtypeStruct((t, h), jnp.float32),
        scratch_types=[
            pltpu.VMEM((half,), jnp.int32),
            pltpu.VMEM((half,), jnp.int32),
            pltpu.VMEM((half, 16), jnp.float32),
            pltpu.VMEM((half, 16), jnp.float32),
            pltpu.VMEM((half, h), jnp.float32),
            pltpu.VMEM((half, h), jnp.float32),
            pltpu.SemaphoreType.DMA,
            pltpu.SemaphoreType.DMA,
        ],
    )
    def k(ys_hbm, p0_hbm, p1_hbm, w1_hbm, w2_hbm, y_hbm,
          pa, pb, w1_v, w2_v, a_v, b_v, sem0, sem1):
        wid = lax.axis_index("s") * _NC + lax.axis_index("c")
        base = wid * tpw

        def do_half(off):
            pltpu.sync_copy(p0_hbm.at[pl.ds(base + off, half)], pa)
            pltpu.sync_copy(p1_hbm.at[pl.ds(base + off, half)], pb)
            pltpu.sync_copy(w1_hbm.at[pl.ds(base + off, half)], w1_v)
            pltpu.sync_copy(w2_hbm.at[pl.ds(base + off, half)], w2_v)
            cp0 = pltpu.async_copy(ys_hbm.at[pa], a_v, sem0)
            cp1 = pltpu.async_copy(ys_hbm.at[pb], b_v, sem1)
            cp0.wait()
            cp1.wait()

            def row(r, carry):
                wa = w1_v[r, :]
                wb = w2_v[r, :]
                for c in range(h // 16):
                    sl = pl.ds(c * 16, 16)
                    a_v[r, sl] = a_v[r, sl] * wa + b_v[r, sl] * wb
                return carry

            lax.fori_loop(0, half, row, 0)
            pltpu.sync_copy(a_v, y_hbm.at[pl.ds(base + off, half)])

        do_half(0)
        do_half(half)

    return k(ys, p0, p1, w1b, w2b)


def _gmm_body(ff, e_ref, b_ref, xs_ref, wg_ref, wu_ref, wd_ref, out_ref):
    j = pl.program_id(1)

    x = xs_ref[...]
    g = lax.dot_general(x, wg_ref[0], (((1,), (1,)), ((), ())),
                        preferred_element_type=jnp.float32)
    u = lax.dot_general(x, wu_ref[0], (((1,), (1,)), ((), ())),
                        preferred_element_type=jnp.float32)
    hh = (g * jax.nn.sigmoid(g)) * u
    cols = j * _BFF + lax.broadcasted_iota(jnp.int32, (1, _BFF), 1)
    hh = jnp.where(cols < ff, hh, 0.0)
    rows_ff = j * _BFF + lax.broadcasted_iota(jnp.int32, (_BFF, 1), 0)
    wd = jnp.where(rows_ff < ff, wd_ref[0], 0.0)
    contrib = jnp.dot(hh, wd, preferred_element_type=jnp.float32)

    @pl.when(j == 0)
    def _():
        out_ref[...] = contrib

    @pl.when(j != 0)
    def _():
        out_ref[...] += contrib


def _gmm(xs, Wgate_t, Wup_t, Wdown, wi_e, wi_b, nw):
    m, h = xs.shape
    ff = Wgate_t.shape[1]
    nff = -(-ff // _BFF)
    grid_spec = pltpu.PrefetchScalarGridSpec(
        num_scalar_prefetch=2,
        grid=(nw, nff),
        in_specs=[
            pl.BlockSpec((_BM, h), lambda i, j, e, b: (b[i], 0)),
            pl.BlockSpec((1, _BFF, h), lambda i, j, e, b: (e[i], j, 0)),
            pl.BlockSpec((1, _BFF, h), lambda i, j, e, b: (e[i], j, 0)),
            pl.BlockSpec((1, _BFF, h), lambda i, j, e, b: (e[i], j, 0)),
        ],
        out_specs=pl.BlockSpec((_BM, h), lambda i, j, e, b: (i, 0)),
    )
    return pl.pallas_call(
        functools.partial(_gmm_body, ff),
        grid_spec=grid_spec,
        out_shape=jax.ShapeDtypeStruct((nw * _BM, h), jnp.float32),
        compiler_params=pltpu.CompilerParams(
            dimension_semantics=("arbitrary", "arbitrary")),
    )(wi_e, wi_b, xs, Wgate_t, Wup_t, Wdown)


def kernel(x, Wg, Wgate, Wup, Wdown):
    b, s, h = x.shape
    e_num = Wg.shape[0]
    xf = x.reshape(-1, h)
    t = xf.shape[0]
    m = t * _TOPK
    nb = m // _BM
    nw = nb + e_num - 1

    a1, a2, w1b, w2b = _gate(xf, Wg)
    eid = jnp.concatenate([a1, a2], axis=1).reshape(-1)

    oh = (eid[:, None] == jnp.arange(e_num)[None, :]).astype(jnp.int32)
    inc = jnp.cumsum(oh, axis=0)
    counts = inc[-1]
    off = jnp.concatenate([jnp.zeros((1,), jnp.int32), jnp.cumsum(counts)])
    rank = jnp.sum((inc - oh) * oh, axis=1)
    ip = (off[eid] + rank).astype(jnp.int32)
    tok_ids = jnp.zeros((m,), jnp.int32).at[ip].set(
        (jnp.arange(m, dtype=jnp.int32) // _TOPK))

    starts_e, ends_e = off[:e_num], off[1:]
    b_start = jnp.arange(nb, dtype=jnp.int32) * _BM
    overlap = ((starts_e[:, None] < (b_start + _BM)[None, :])
               & (ends_e[:, None] > b_start[None, :])
               & (counts[:, None] > 0))
    sel = jnp.nonzero(overlap.reshape(-1), size=nw,
                      fill_value=nb * e_num - 1)[0].astype(jnp.int32)
    wi_e = sel // nb
    wi_b = sel % nb

    fb = starts_e // _BM
    lb = jnp.maximum(ends_e - 1, 0) // _BM
    nblk = jnp.where(counts > 0, lb - fb + 1, 0)
    base_items = jnp.cumsum(nblk) - nblk
    item = base_items[eid] + (ip // _BM - fb[eid])
    ep = (item * _BM + ip % _BM).astype(jnp.int32)
    p0, p1 = ep[0::2], ep[1::2]

    xs = _dispatch(xf, tok_ids)

    ys = _gmm(xs, jnp.swapaxes(Wgate, 1, 2), jnp.swapaxes(Wup, 1, 2),
              Wdown, wi_e, wi_b, nw)

    y = _combine(ys, p0, p1, w1b, w2b)
    return y.reshape(b, s, h)


# speedup vs baseline: 1.5726x; 1.0481x over previous
import functools

import jax
import jax.numpy as jnp
from jax import lax
from jax.experimental import pallas as pl
from jax.experimental.pallas import tpu as pltpu
from jax.experimental.pallas import tpu_sc as plsc

_TOPK = 2
_BM = 256
_BFF = 1408
_NC = 2
_NS = 16
_NWRK = _NC * _NS


def _gate_body(x_ref, wg_ref, i1_ref, i2_ref, w1_ref, w2_ref):
    x = x_ref[...]
    logits = lax.dot_general(x, wg_ref[...], (((1,), (1,)), ((), ())),
                             preferred_element_type=jnp.float32)
    m = jnp.max(logits, axis=-1, keepdims=True)
    ex = jnp.exp(logits - m)
    p = ex / jnp.sum(ex, axis=-1, keepdims=True)
    e_num = p.shape[-1]
    idxs = lax.broadcasted_iota(jnp.int32, p.shape, 1)
    m1 = jnp.max(p, axis=-1, keepdims=True)
    a1 = jnp.min(jnp.where(p == m1, idxs, e_num), axis=-1, keepdims=True)
    p2 = jnp.where(idxs == a1, -1.0, p)
    m2 = jnp.max(p2, axis=-1, keepdims=True)
    a2 = jnp.min(jnp.where(p2 == m2, idxs, e_num), axis=-1, keepdims=True)
    s = m1 + m2 + 1e-20
    i1_ref[...] = a1
    i2_ref[...] = a2
    w1_ref[...] = jnp.broadcast_to(m1 / s, w1_ref.shape)
    w2_ref[...] = jnp.broadcast_to(m2 / s, w2_ref.shape)


def _gate(xf, Wg):
    t = xf.shape[0]
    return pl.pallas_call(
        _gate_body,
        out_shape=[
            jax.ShapeDtypeStruct((t, 1), jnp.int32),
            jax.ShapeDtypeStruct((t, 1), jnp.int32),
            jax.ShapeDtypeStruct((t, 16), jnp.float32),
            jax.ShapeDtypeStruct((t, 16), jnp.float32),
        ],
    )(xf, Wg)


def _dispatch(xf, pe, po):
    t, h = xf.shape
    m = t * _TOPK
    tpw = t // _NWRK
    mesh = plsc.VectorSubcoreMesh(core_axis_name="c", subcore_axis_name="s")

    @functools.partial(
        pl.kernel, mesh=mesh,
        out_type=jax.ShapeDtypeStruct((m, h), jnp.float32),
        scratch_types=[
            pltpu.VMEM((tpw,), jnp.int32),
            pltpu.VMEM((tpw,), jnp.int32),
            pltpu.VMEM((tpw, h), jnp.float32),
            pltpu.SemaphoreType.DMA,
            pltpu.SemaphoreType.DMA,
        ],
    )
    def k(xf_hbm, pe_hbm, po_hbm, xs_hbm, pa, pb, buf, sem0, sem1):
        wid = lax.axis_index("s") * _NC + lax.axis_index("c")
        base = wid * tpw
        pltpu.sync_copy(pe_hbm.at[pl.ds(base, tpw)], pa)
        pltpu.sync_copy(po_hbm.at[pl.ds(base, tpw)], pb)
        pltpu.sync_copy(xf_hbm.at[pl.ds(base, tpw)], buf)
        cp0 = pltpu.async_copy(buf, xs_hbm.at[pa], sem0)
        cp1 = pltpu.async_copy(buf, xs_hbm.at[pb], sem1)
        cp0.wait()
        cp1.wait()

    return k(xf, pe, po)


def _combine(ys, p0, p1, w1b, w2b):
    m, h = ys.shape
    t = p0.shape[0]
    tpw = t // _NWRK
    half = tpw // 2
    mesh = plsc.VectorSubcoreMesh(core_axis_name="c", subcore_axis_name="s")

    @functools.partial(
        pl.kernel, mesh=mesh,
        out_type=jax.ShapeDtypeStruct((t, h), jnp.float32),
        scratch_types=[
            pltpu.VMEM((half,), jnp.int32),
            pltpu.VMEM((half,), jnp.int32),
            pltpu.VMEM((half, 16), jnp.float32),
            pltpu.VMEM((half, 16), jnp.float32),
            pltpu.VMEM((half, h), jnp.float32),
            pltpu.VMEM((half, h), jnp.float32),
            pltpu.SemaphoreType.DMA,
            pltpu.SemaphoreType.DMA,
        ],
    )
    def k(ys_hbm, p0_hbm, p1_hbm, w1_hbm, w2_hbm, y_hbm,
          pa, pb, w1_v, w2_v, a_v, b_v, sem0, sem1):
        wid = lax.axis_index("s") * _NC + lax.axis_index("c")
        base = wid * tpw

        def do_half(off):
            pltpu.sync_copy(p0_hbm.at[pl.ds(base + off, half)], pa)
            pltpu.sync_copy(p1_hbm.at[pl.ds(base + off, half)], pb)
            pltpu.sync_copy(w1_hbm.at[pl.ds(base + off, half)], w1_v)
            pltpu.sync_copy(w2_hbm.at[pl.ds(base + off, half)], w2_v)
            cp0 = pltpu.async_copy(ys_hbm.at[pa], a_v, sem0)
            cp1 = pltpu.async_copy(ys_hbm.at[pb], b_v, sem1)
            cp0.wait()
            cp1.wait()

            def row(r, carry):
                wa = w1_v[r, :]
                wb = w2_v[r, :]
                for c in range(h // 16):
                    sl = pl.ds(c * 16, 16)
                    a_v[r, sl] = a_v[r, sl] * wa + b_v[r, sl] * wb
                return carry

            lax.fori_loop(0, half, row, 0)
            pltpu.sync_copy(a_v, y_hbm.at[pl.ds(base + off, half)])

        do_half(0)
        do_half(half)

    return k(ys, p0, p1, w1b, w2b)


def _gmm_body(ff, e_ref, b_ref, xs_ref, wg_ref, wu_ref, wd_ref, out_ref):
    j = pl.program_id(1)

    x = xs_ref[...]
    g = lax.dot_general(x, wg_ref[0], (((1,), (1,)), ((), ())),
                        preferred_element_type=jnp.float32)
    u = lax.dot_general(x, wu_ref[0], (((1,), (1,)), ((), ())),
                        preferred_element_type=jnp.float32)
    hh = (g * jax.nn.sigmoid(g)) * u
    cols = j * _BFF + lax.broadcasted_iota(jnp.int32, (1, _BFF), 1)
    hh = jnp.where(cols < ff, hh, 0.0)
    rows_ff = j * _BFF + lax.broadcasted_iota(jnp.int32, (_BFF, 1), 0)
    wd = jnp.where(rows_ff < ff, wd_ref[0], 0.0)
    contrib = jnp.dot(hh, wd, preferred_element_type=jnp.float32)

    @pl.when(j == 0)
    def _():
        out_ref[...] = contrib

    @pl.when(j != 0)
    def _():
        out_ref[...] += contrib


def _gmm(xs, Wgate_t, Wup_t, Wdown, wi_e, wi_b, nw):
    m, h = xs.shape
    ff = Wgate_t.shape[1]
    nff = -(-ff // _BFF)
    grid_spec = pltpu.PrefetchScalarGridSpec(
        num_scalar_prefetch=2,
        grid=(nw, nff),
        in_specs=[
            pl.BlockSpec((_BM, h), lambda i, j, e, b: (b[i], 0)),
            pl.BlockSpec((1, _BFF, h), lambda i, j, e, b: (e[i], j, 0)),
            pl.BlockSpec((1, _BFF, h), lambda i, j, e, b: (e[i], j, 0)),
            pl.BlockSpec((1, _BFF, h), lambda i, j, e, b: (e[i], j, 0)),
        ],
        out_specs=pl.BlockSpec((_BM, h), lambda i, j, e, b: (i, 0)),
    )
    return pl.pallas_call(
        functools.partial(_gmm_body, ff),
        grid_spec=grid_spec,
        out_shape=jax.ShapeDtypeStruct((nw * _BM, h), jnp.float32),
        compiler_params=pltpu.CompilerParams(
            dimension_semantics=("arbitrary", "arbitrary")),
    )(wi_e, wi_b, xs, Wgate_t, Wup_t, Wdown)


def kernel(x, Wg, Wgate, Wup, Wdown):
    b, s, h = x.shape
    e_num = Wg.shape[0]
    xf = x.reshape(-1, h)
    t = xf.shape[0]
    m = t * _TOPK
    nb = m // _BM
    nw = nb + e_num - 1

    a1, a2, w1b, w2b = _gate(xf, Wg)
    eid = jnp.concatenate([a1, a2], axis=1).reshape(-1)

    oh = (eid[:, None] == jnp.arange(e_num)[None, :]).astype(jnp.int32)
    inc = jnp.cumsum(oh, axis=0)
    counts = inc[-1]
    off = jnp.concatenate([jnp.zeros((1,), jnp.int32), jnp.cumsum(counts)])
    rank = jnp.sum((inc - oh) * oh, axis=1)
    ip = (off[eid] + rank).astype(jnp.int32)

    starts_e, ends_e = off[:e_num], off[1:]
    b_start = jnp.arange(nb, dtype=jnp.int32) * _BM
    overlap = ((starts_e[:, None] < (b_start + _BM)[None, :])
               & (ends_e[:, None] > b_start[None, :])
               & (counts[:, None] > 0))
    sel = jnp.nonzero(overlap.reshape(-1), size=nw,
                      fill_value=nb * e_num - 1)[0].astype(jnp.int32)
    wi_e = sel // nb
    wi_b = sel % nb

    fb = starts_e // _BM
    lb = jnp.maximum(ends_e - 1, 0) // _BM
    nblk = jnp.where(counts > 0, lb - fb + 1, 0)
    base_items = jnp.cumsum(nblk) - nblk
    item = base_items[eid] + (ip // _BM - fb[eid])
    ep = (item * _BM + ip % _BM).astype(jnp.int32)
    p0, p1 = ep[0::2], ep[1::2]

    xs = _dispatch(xf, ip[0::2], ip[1::2])

    ys = _gmm(xs, jnp.swapaxes(Wgate, 1, 2), jnp.swapaxes(Wup, 1, 2),
              Wdown, wi_e, wi_b, nw)

    y = _combine(ys, p0, p1, w1b, w2b)
    return y.reshape(b, s, h)
